# baseline (speedup 1.0000x reference)
import functools

import jax
import jax.numpy as jnp
from jax import lax
from jax.experimental import pallas as pl
from jax.experimental.pallas import tpu as pltpu
from jax.experimental.pallas import tpu_sc as plsc

B, S, H = 1, 2048, 2048
I = 1024
E = 8
K = 2
P = S * K
T = 128
NT = P // T
NITEMS = NT + E - 1

NC, NS = 2, 16
NW = NC * NS


def _routing_body(x_ref, gw_ref, bias_ref, pos0_ref, pos1_ref, p0_ref, p1_ref,
                  counts_ref, mv_ref):
    xf = x_ref[...]
    scores = lax.dot_general(
        xf, gw_ref[...], (((1,), (1,)), ((), ())),
        preferred_element_type=jnp.float32)
    biased = scores + bias_ref[...]
    iota = lax.broadcasted_iota(jnp.int32, (S, E), 1)
    neg_inf = jnp.float32(-jnp.inf)

    v1 = jnp.max(biased, axis=1, keepdims=True)
    idx1 = jnp.min(jnp.where(biased == v1, iota, E), axis=1, keepdims=True)
    masked = jnp.where(iota == idx1, neg_inf, biased)
    v2 = jnp.max(masked, axis=1, keepdims=True)
    idx2 = jnp.min(jnp.where(masked == v2, iota, E), axis=1, keepdims=True)

    u1 = jnp.max(scores, axis=1, keepdims=True)
    uidx1 = jnp.min(jnp.where(scores == u1, iota, E), axis=1, keepdims=True)
    u2 = jnp.max(jnp.where(iota == uidx1, neg_inf, scores), axis=1,
                 keepdims=True)
    p1 = jax.nn.sigmoid(u1)
    p2 = jax.nn.sigmoid(u2)
    ps = p1 + p2
    p0_ref[...] = p1 / ps
    p1_ref[...] = p2 / ps

    oh1 = (iota == idx1).astype(jnp.float32)
    oh2 = (iota == idx2).astype(jnp.float32)

    counts = jnp.sum(oh1 + oh2, axis=0, keepdims=True)
    counts_ref[...] = counts
    freq = counts / jnp.float32(P)
    fmean = jnp.sum(freq) / jnp.float32(E)
    mv_ref[...] = jnp.full((1, 1), (jnp.max(freq) - fmean) / fmean,
                           jnp.float32)

    tri = (lax.broadcasted_iota(jnp.int32, (S, S), 1)
           < lax.broadcasted_iota(jnp.int32, (S, S), 0)).astype(jnp.bfloat16)
    c1ex = lax.dot_general(tri, oh1.astype(jnp.bfloat16),
                           (((1,), (0,)), ((), ())),
                           preferred_element_type=jnp.float32)
    c2ex = lax.dot_general(tri, oh2.astype(jnp.bfloat16),
                           (((1,), (0,)), ((), ())),
                           preferred_element_type=jnp.float32)
    lt1 = (idx1 < iota).astype(jnp.float32)
    lt2 = (idx2 < iota).astype(jnp.float32)
    offsets = jnp.sum(lt1 + lt2, axis=0, keepdims=True)
    c1tot = jnp.sum(oh1, axis=0, keepdims=True)
    pos0_ref[...] = jnp.sum((offsets + c1ex) * oh1, axis=1, keepdims=True)
    pos1_ref[...] = jnp.sum((offsets + c1tot + c2ex) * oh2, axis=1,
                            keepdims=True)


def _shared_body(x_ref, sg_ref, su_ref, sd_ref, out_ref):
    g = lax.dot_general(x_ref[...], sg_ref[...], (((1,), (1,)), ((), ())),
                        preferred_element_type=jnp.float32)
    u = lax.dot_general(x_ref[...], su_ref[...], (((1,), (1,)), ((), ())),
                        preferred_element_type=jnp.float32)
    h = (jax.nn.silu(g) * u).astype(jnp.bfloat16)
    out_ref[...] = lax.dot_general(h, sd_ref[...], (((1,), (1,)), ((), ())),
                                   preferred_element_type=jnp.float32)


def _grouped_body(tile_s, exp_s, lo_s, hi_s, x_ref, wg_ref, wu_ref, wd_ref,
                  y_ref):
    i = pl.program_id(0)
    lo = lo_s[i]
    hi = hi_s[i]
    tile = tile_s[i]

    @pl.when(hi > lo)
    def _work():
        xb = x_ref[...].astype(jnp.bfloat16)
        g = lax.dot_general(xb, wg_ref[0], (((1,), (1,)), ((), ())),
                            preferred_element_type=jnp.float32)
        u = lax.dot_general(xb, wu_ref[0], (((1,), (1,)), ((), ())),
                            preferred_element_type=jnp.float32)
        h = (jax.nn.silu(g) * u).astype(jnp.bfloat16)
        y = lax.dot_general(h, wd_ref[0], (((1,), (1,)), ((), ())),
                            preferred_element_type=jnp.float32)
        rows = tile * T + lax.broadcasted_iota(jnp.int32, (T, 1), 0)
        m = ((rows >= lo) & (rows < hi)).astype(jnp.float32)
        contrib = y * m

        @pl.when(lo == tile * T)
        def _init():
            y_ref[...] = contrib

        @pl.when(lo != tile * T)
        def _acc():
            y_ref[...] += contrib


def _mesh():
    return plsc.VectorSubcoreMesh(core_axis_name="c", subcore_axis_name="s",
                                  num_cores=NC, num_subcores=NS)


@functools.cache
def _build_sc_dispatch():
    return functools.partial(
        pl.kernel,
        out_type=jax.ShapeDtypeStruct((P, H), jnp.float32),
        mesh=_mesh(),
        scratch_types=[
            pltpu.VMEM((32,), jnp.int32),
            pltpu.VMEM((32, H), jnp.float32),
            pltpu.SemaphoreType.DMA,
        ],
    )(_sc_dispatch_body)


def _sc_dispatch(xf, pos):
    return _build_sc_dispatch()(xf, pos)


def _sc_dispatch_body(xf_hbm, pos_hbm, xs_hbm, idx_v, row_v, sem):
    wid = lax.axis_index("s") * NC + lax.axis_index("c")
    k = wid // 16
    tb = (wid % 16) * 128
    for c in range(4):
        base = tb + 32 * c
        pltpu.sync_copy(pos_hbm.at[k, pl.ds(base, 32)], idx_v)
        pltpu.sync_copy(xf_hbm.at[pl.ds(base, 32)], row_v)
        pltpu.async_copy(row_v, xs_hbm.at[idx_v], sem).wait()


@functools.cache
def _build_sc_combine():
    return functools.partial(
        pl.kernel,
        out_type=jax.ShapeDtypeStruct((S, H), jnp.float32),
        mesh=_mesh(),
        scratch_types=[
            pltpu.VMEM((16,), jnp.int32),
            pltpu.VMEM((16,), jnp.int32),
            pltpu.VMEM((16, 16), jnp.float32),
            pltpu.VMEM((16, 16), jnp.float32),
            pltpu.VMEM((16, H), jnp.float32),
            pltpu.VMEM((16, H), jnp.float32),
            pltpu.VMEM((16, H), jnp.float32),
            pltpu.SemaphoreType.DMA,
        ],
    )(_sc_combine_body)


def _sc_combine(shared, y, pos, pw):
    return _build_sc_combine()(shared, y, pos, pw)


def _sc_combine_body(sh_hbm, y_hbm, pos_hbm, pw_hbm, out_hbm,
                     i0_v, i1_v, p0_v, p1_v, y0_v, y1_v, s_v, sem):
    wid = lax.axis_index("s") * NC + lax.axis_index("c")
    for c in range(4):
        base = wid * 64 + 16 * c
        pltpu.sync_copy(pos_hbm.at[0, pl.ds(base, 16)], i0_v)
        pltpu.sync_copy(pos_hbm.at[1, pl.ds(base, 16)], i1_v)
        pltpu.sync_copy(pw_hbm.at[0, pl.ds(base, 16)], p0_v)
        pltpu.sync_copy(pw_hbm.at[1, pl.ds(base, 16)], p1_v)
        a = pltpu.async_copy(y_hbm.at[i0_v], y0_v, sem)
        b = pltpu.async_copy(y_hbm.at[i1_v], y1_v, sem)
        pltpu.sync_copy(sh_hbm.at[pl.ds(base, 16)], s_v)
        a.wait()
        b.wait()

        def tok(t, _):
            p0b = p0_v[t, :]
            p1b = p1_v[t, :]

            def col(j, _):
                d = pl.ds(j * 16, 16)
                s_v[t, d] = s_v[t, d] + p0b * y0_v[t, d] + p1b * y1_v[t, d]
                return 0

            lax.fori_loop(0, H // 16, col, 0, unroll=8)
            return 0

        lax.fori_loop(0, 16, tok, 0)
        pltpu.sync_copy(s_v, out_hbm.at[pl.ds(base, 16)])


def _make_schedule(counts):
    counts_i = counts[0].astype(jnp.int32)
    offs = jnp.concatenate(
        [jnp.zeros((1,), jnp.int32), jnp.cumsum(counts_i)])
    bp = jnp.sort(jnp.concatenate(
        [jnp.arange(NT, dtype=jnp.int32) * T, offs[1:E]]))
    nxt = jnp.concatenate([bp[1:], jnp.array([P], jnp.int32)])
    item_tile = jnp.clip(bp // T, 0, NT - 1)
    item_expert = jnp.clip(
        jnp.searchsorted(offs, bp, side="right") - 1, 0, E - 1
    ).astype(jnp.int32)
    return item_tile, item_expert, bp, nxt


@jax.jit
def kernel(x, gate_w, e_bias, wg, wu, wd, sg, su, sd):
    xf = x.reshape(S, H)
    x_bf = xf.astype(jnp.bfloat16)

    pos0, pos1, p0, p1, counts, mv = pl.pallas_call(
        _routing_body,
        out_shape=(
            jax.ShapeDtypeStruct((S, 1), jnp.float32),
            jax.ShapeDtypeStruct((S, 1), jnp.float32),
            jax.ShapeDtypeStruct((S, 1), jnp.float32),
            jax.ShapeDtypeStruct((S, 1), jnp.float32),
            jax.ShapeDtypeStruct((1, E), jnp.float32),
            jax.ShapeDtypeStruct((1, 1), jnp.float32),
        ),
    )(xf, gate_w, e_bias.reshape(1, E))

    pos = jnp.stack([pos0[:, 0], pos1[:, 0]]).astype(jnp.int32)
    pw = jnp.broadcast_to(
        jnp.stack([p0[:, 0], p1[:, 0]])[:, :, None], (K, S, 16))
    item_tile, item_expert, item_lo, item_hi = _make_schedule(counts)

    x_sorted = _sc_dispatch(xf, pos)

    shared = pl.pallas_call(
        _shared_body,
        grid=(2,),
        in_specs=[
            pl.BlockSpec((S // 2, H), lambda t: (t, 0)),
            pl.BlockSpec((I, H), lambda t: (0, 0)),
            pl.BlockSpec((I, H), lambda t: (0, 0)),
            pl.BlockSpec((H, I), lambda t: (0, 0)),
        ],
        out_specs=pl.BlockSpec((S // 2, H), lambda t: (t, 0)),
        out_shape=jax.ShapeDtypeStruct((S, H), jnp.float32),
    )(x_bf, sg.astype(jnp.bfloat16), su.astype(jnp.bfloat16),
      sd.astype(jnp.bfloat16))

    wg_bf = wg.astype(jnp.bfloat16)
    wu_bf = wu.astype(jnp.bfloat16)
    wd_bf = wd.astype(jnp.bfloat16)

    y_sorted = pl.pallas_call(
        _grouped_body,
        grid_spec=pltpu.PrefetchScalarGridSpec(
            num_scalar_prefetch=4,
            grid=(NITEMS,),
            in_specs=[
                pl.BlockSpec((T, H), lambda i, ts, es, ls, hs: (ts[i], 0)),
                pl.BlockSpec((1, I, H),
                             lambda i, ts, es, ls, hs: (es[i], 0, 0)),
                pl.BlockSpec((1, I, H),
                             lambda i, ts, es, ls, hs: (es[i], 0, 0)),
                pl.BlockSpec((1, H, I),
                             lambda i, ts, es, ls, hs: (es[i], 0, 0)),
            ],
            out_specs=pl.BlockSpec((T, H),
                                   lambda i, ts, es, ls, hs: (ts[i], 0)),
        ),
        out_shape=jax.ShapeDtypeStruct((P, H), jnp.float32),
    )(item_tile, item_expert, item_lo, item_hi, x_sorted, wg_bf, wu_bf, wd_bf)

    out = _sc_combine(shared, y_sorted, pos, pw)

    return (out.reshape(B, S, H), jnp.float32(0.0), mv[0, 0])
